# baseline (speedup 1.0000x reference)
import jax
import jax.numpy as jnp
from jax import lax
from jax.experimental import pallas as pl
from jax.experimental.pallas import tpu as pltpu
from jax.experimental.pallas import tpu_sc as plsc

_NEG_INF = -10.0 ** 10
_B = 4096
_V = 100000
_SUB = 8
_LANES = 16
_GROUPS = 8
_WCOLS = 128
_NUM_CORES = 2
_NUM_SUBCORES = 16
_NW = _NUM_CORES * _NUM_SUBCORES
_NB_SLOT = 25
_SLOT_ROWS = _NB_SLOT * _SUB
_NSLOT = _V // _SLOT_ROWS
_NBUF = 5
_NROUND = _NSLOT // _NBUF

_GATHER_1D = lax.GatherDimensionNumbers(
    offset_dims=(), collapsed_slice_dims=(0,), start_index_map=(0,))


def _permute(x, idx):
    return lax.gather(
        x, idx.reshape(_LANES, 1), dimension_numbers=_GATHER_1D,
        slice_sizes=(1,), mode=lax.GatherScatterMode.PROMISE_IN_BOUNDS)


def _xlane_max(x, lane):
    for s in (8, 4, 2, 1):
        x = jnp.maximum(x, _permute(x, lane ^ s))
    return x


def _sc_body(pred_ref, label_ref, out_ref, label_v, accb, fyb,
             b0, b1, b2, b3, b4, sums_v, sems):
    bufs = (b0, b1, b2, b3, b4)
    wid = lax.axis_index("s") * _NUM_CORES + lax.axis_index("c")
    col0 = pl.multiple_of(wid * _WCOLS, _WCOLS)

    pltpu.sync_copy(label_ref.at[pl.ds(col0, _WCOLS)], label_v)
    for g in range(_GROUPS):
        accb[pl.ds(g * _LANES, _LANES)] = jnp.full(
            (_LANES,), _NEG_INF, jnp.float32)
        fyb[pl.ds(g * _LANES, _LANES)] = jnp.full(
            (_LANES,), _NEG_INF, jnp.float32)

    def start_slot(s, u):
        pltpu.async_copy(
            pred_ref.at[pl.ds(pl.multiple_of(s * _SLOT_ROWS, _SUB),
                              _SLOT_ROWS),
                        pl.ds(col0, _WCOLS)],
            bufs[u], sems.at[u])

    def wait_slot(u):
        pltpu.make_async_copy(
            pred_ref.at[pl.ds(0, _SLOT_ROWS), pl.ds(0, _WCOLS)],
            bufs[u], sems.at[u]).wait()

    for u in range(_NBUF):
        start_slot(u, u)

    def process_slot(s, u):
        lane = lax.broadcasted_iota(jnp.int32, (_LANES,), 0)
        negv = jnp.full((_LANES,), _NEG_INF, jnp.float32)
        wait_slot(u)
        buf = bufs[u]
        base = s * _SLOT_ROWS
        lblv = [label_v[pl.ds(g * _LANES, _LANES)] for g in range(_GROUPS)]

        hitv = jnp.zeros((_LANES,), jnp.float32)
        for g in range(_GROUPS):
            loc = lblv[g] - base
            inb = (loc >= 0) & (loc < _SLOT_ROWS)
            hitv = hitv + jnp.where(inb, 1.0, 0.0)
        hit = _xlane_max(hitv, lane)[0] > 0.5

        @pl.when(jnp.logical_not(hit))
        def _():
            accs = [accb[pl.ds(g * _LANES, _LANES)] for g in range(_GROUPS)]

            @plsc.parallel_loop(0, _SLOT_ROWS, 1, unroll=2,
                                carry=tuple(accs))
            def red(row, acc):
                return tuple(
                    jnp.maximum(a, buf[row, pl.ds(g * _LANES, _LANES)])
                    for g, a in enumerate(acc))

            for g in range(_GROUPS):
                accb[pl.ds(g * _LANES, _LANES)] = red[g]

        @pl.when(hit)
        def _():
            accs = [accb[pl.ds(g * _LANES, _LANES)] for g in range(_GROUPS)]
            fys = [fyb[pl.ds(g * _LANES, _LANES)] for g in range(_GROUPS)]

            @plsc.parallel_loop(0, _SLOT_ROWS, 1,
                                carry=tuple(accs) + tuple(fys))
            def red(row, c):
                out = list(c[:_GROUPS])
                fy = list(c[_GROUPS:])
                rowv = jnp.full((_LANES,), base + row, jnp.int32)
                for g in range(_GROUPS):
                    x = buf[row, pl.ds(g * _LANES, _LANES)]
                    isl = lblv[g] == rowv
                    out[g] = jnp.maximum(out[g], jnp.where(isl, negv, x))
                    fy[g] = jnp.maximum(fy[g], jnp.where(isl, x, negv))
                return tuple(out) + tuple(fy)

            for g in range(_GROUPS):
                accb[pl.ds(g * _LANES, _LANES)] = red[g]
                fyb[pl.ds(g * _LANES, _LANES)] = red[_GROUPS + g]

    def round_body(ri, carry):
        for u in range(_NBUF):
            s = ri * _NBUF + u
            process_slot(s, u)

            @pl.when(s < _NSLOT - _NBUF)
            def _():
                start_slot(s + _NBUF, u)
        return carry

    lax.fori_loop(0, _NROUND, round_body, 0)

    hsum = jnp.zeros((_LANES,), jnp.float32)
    for g in range(_GROUPS):
        fnym = accb[pl.ds(g * _LANES, _LANES)]
        fy = fyb[pl.ds(g * _LANES, _LANES)]
        hsum = hsum + (jnp.maximum(2.0 - fy, 0.0)
                       + jnp.maximum(1.0 + fnym, 0.0))
    sums_v[...] = hsum
    pltpu.sync_copy(sums_v, out_ref.at[wid])


@jax.jit
def _sc_loss(pred_t, lbl):
    mesh = plsc.VectorSubcoreMesh(
        core_axis_name="c", subcore_axis_name="s",
        num_cores=_NUM_CORES, num_subcores=_NUM_SUBCORES)
    part = pl.kernel(
        _sc_body,
        out_type=jax.ShapeDtypeStruct((_NW, _LANES), jnp.float32),
        mesh=mesh,
        compiler_params=pltpu.CompilerParams(use_tc_tiling_on_sc=True),
        scratch_types=(
            [pltpu.VMEM((_WCOLS,), jnp.int32),
             pltpu.VMEM((_WCOLS,), jnp.float32),
             pltpu.VMEM((_WCOLS,), jnp.float32)]
            + [pltpu.VMEM((_SLOT_ROWS, _WCOLS), jnp.float32)] * _NBUF
            + [pltpu.VMEM((_LANES,), jnp.float32),
               pltpu.SemaphoreType.DMA((_NBUF,))]
        ),
    )(pred_t, lbl)
    return jnp.sum(part) * (1.0 / _B)


def kernel(prediction, label):
    return _sc_loss(prediction.T, label.astype(jnp.int32))


# speedup vs baseline: 1.0676x; 1.0676x over previous
import jax
import jax.numpy as jnp
from jax import lax
from jax.experimental import pallas as pl
from jax.experimental.pallas import tpu as pltpu
from jax.experimental.pallas import tpu_sc as plsc

_NEG_INF = -10.0 ** 10
_B = 4096
_V = 100000
_SUB = 8
_LANES = 16
_GROUPS = 8
_WCOLS = 128
_NUM_CORES = 2
_NUM_SUBCORES = 16
_NW = _NUM_CORES * _NUM_SUBCORES
_NB_SLOT = 25
_SLOT_ROWS = _NB_SLOT * _SUB
_V_SC = 60000
_V_TC = _V - _V_SC
_NSLOT = _V_SC // _SLOT_ROWS
_NBUF = 4
_NROUND = _NSLOT // _NBUF
_TC_VB = 800
_TC_NV = _V_TC // _TC_VB

_GATHER_1D = lax.GatherDimensionNumbers(
    offset_dims=(), collapsed_slice_dims=(0,), start_index_map=(0,))


def _permute(x, idx):
    return lax.gather(
        x, idx.reshape(_LANES, 1), dimension_numbers=_GATHER_1D,
        slice_sizes=(1,), mode=lax.GatherScatterMode.PROMISE_IN_BOUNDS)


def _xlane_max(x, lane):
    for s in (8, 4, 2, 1):
        x = jnp.maximum(x, _permute(x, lane ^ s))
    return x


def _sc_body(pred_ref, label_ref, fn_ref, fy_ref, label_v, accb, fyb,
             b0, b1, b2, b3, sems):
    bufs = (b0, b1, b2, b3)
    wid = lax.axis_index("s") * _NUM_CORES + lax.axis_index("c")
    col0 = pl.multiple_of(wid * _WCOLS, _WCOLS)

    pltpu.sync_copy(label_ref.at[pl.ds(col0, _WCOLS)], label_v)
    for g in range(_GROUPS):
        accb[pl.ds(g * _LANES, _LANES)] = jnp.full(
            (_LANES,), _NEG_INF, jnp.float32)
        fyb[pl.ds(g * _LANES, _LANES)] = jnp.full(
            (_LANES,), _NEG_INF, jnp.float32)

    def start_slot(s, u):
        pltpu.async_copy(
            pred_ref.at[pl.ds(pl.multiple_of(s * _SLOT_ROWS, _SUB),
                              _SLOT_ROWS),
                        pl.ds(col0, _WCOLS)],
            bufs[u], sems.at[u])

    def wait_slot(u):
        pltpu.make_async_copy(
            pred_ref.at[pl.ds(0, _SLOT_ROWS), pl.ds(0, _WCOLS)],
            bufs[u], sems.at[u]).wait()

    for u in range(_NBUF):
        start_slot(u, u)

    def process_slot(s, u):
        lane = lax.broadcasted_iota(jnp.int32, (_LANES,), 0)
        negv = jnp.full((_LANES,), _NEG_INF, jnp.float32)
        wait_slot(u)
        buf = bufs[u]
        base = s * _SLOT_ROWS
        lblv = [label_v[pl.ds(g * _LANES, _LANES)] for g in range(_GROUPS)]

        hitv = jnp.zeros((_LANES,), jnp.float32)
        for g in range(_GROUPS):
            loc = lblv[g] - base
            inb = (loc >= 0) & (loc < _SLOT_ROWS)
            hitv = hitv + jnp.where(inb, 1.0, 0.0)
        hit = _xlane_max(hitv, lane)[0] > 0.5

        @pl.when(jnp.logical_not(hit))
        def _():
            accs = [accb[pl.ds(g * _LANES, _LANES)] for g in range(_GROUPS)]

            @plsc.parallel_loop(0, _SLOT_ROWS, 1, unroll=2,
                                carry=tuple(accs))
            def red(row, acc):
                return tuple(
                    jnp.maximum(a, buf[row, pl.ds(g * _LANES, _LANES)])
                    for g, a in enumerate(acc))

            for g in range(_GROUPS):
                accb[pl.ds(g * _LANES, _LANES)] = red[g]

        @pl.when(hit)
        def _():
            accs = [accb[pl.ds(g * _LANES, _LANES)] for g in range(_GROUPS)]
            fys = [fyb[pl.ds(g * _LANES, _LANES)] for g in range(_GROUPS)]

            @plsc.parallel_loop(0, _SLOT_ROWS, 1,
                                carry=tuple(accs) + tuple(fys))
            def red(row, c):
                out = list(c[:_GROUPS])
                fy = list(c[_GROUPS:])
                rowv = jnp.full((_LANES,), base + row, jnp.int32)
                for g in range(_GROUPS):
                    x = buf[row, pl.ds(g * _LANES, _LANES)]
                    isl = lblv[g] == rowv
                    out[g] = jnp.maximum(out[g], jnp.where(isl, negv, x))
                    fy[g] = jnp.maximum(fy[g], jnp.where(isl, x, negv))
                return tuple(out) + tuple(fy)

            for g in range(_GROUPS):
                accb[pl.ds(g * _LANES, _LANES)] = red[g]
                fyb[pl.ds(g * _LANES, _LANES)] = red[_GROUPS + g]

    def round_body(ri, carry):
        for u in range(_NBUF):
            s = ri * _NBUF + u
            process_slot(s, u)

            @pl.when(s < _NSLOT - _NBUF)
            def _():
                start_slot(s + _NBUF, u)
        return carry

    lax.fori_loop(0, _NROUND, round_body, 0)

    pltpu.sync_copy(accb, fn_ref.at[wid])
    pltpu.sync_copy(fyb, fy_ref.at[wid])


def _tc_body(pred_ref, lbl_ref, fn_ref, fy_ref):
    i = pl.program_id(0)
    x = pred_ref[...]
    lbl = lbl_ref[...]
    rows = (lax.broadcasted_iota(jnp.int32, (_TC_VB, _B), 0)
            + (_V_SC + i * _TC_VB))
    isl = rows == lbl
    fn = jnp.max(jnp.where(isl, _NEG_INF, x), axis=0, keepdims=True)
    fy = jnp.max(jnp.where(isl, x, _NEG_INF), axis=0, keepdims=True)

    @pl.when(i == 0)
    def _():
        fn_ref[...] = fn
        fy_ref[...] = fy

    @pl.when(i > 0)
    def _():
        fn_ref[...] = jnp.maximum(fn_ref[...], fn)
        fy_ref[...] = jnp.maximum(fy_ref[...], fy)


def _combine_body(fn_sc_ref, fy_sc_ref, fn_tc_ref, fy_tc_ref, out_ref):
    fn = jnp.maximum(fn_sc_ref[...], fn_tc_ref[...])
    fy = jnp.maximum(fy_sc_ref[...], fy_tc_ref[...])
    l = jnp.maximum(2.0 - fy, 0.0) + jnp.maximum(1.0 + fn, 0.0)
    out_ref[...] = jnp.sum(l).reshape(1, 1) * (1.0 / _B)


@jax.jit
def _sc_loss(pred_t, lbl):
    mesh = plsc.VectorSubcoreMesh(
        core_axis_name="c", subcore_axis_name="s",
        num_cores=_NUM_CORES, num_subcores=_NUM_SUBCORES)
    fn_sc, fy_sc = pl.kernel(
        _sc_body,
        out_type=(jax.ShapeDtypeStruct((_NW, _WCOLS), jnp.float32),
                  jax.ShapeDtypeStruct((_NW, _WCOLS), jnp.float32)),
        mesh=mesh,
        compiler_params=pltpu.CompilerParams(use_tc_tiling_on_sc=True),
        scratch_types=(
            [pltpu.VMEM((_WCOLS,), jnp.int32),
             pltpu.VMEM((_WCOLS,), jnp.float32),
             pltpu.VMEM((_WCOLS,), jnp.float32)]
            + [pltpu.VMEM((_SLOT_ROWS, _WCOLS), jnp.float32)] * _NBUF
            + [pltpu.SemaphoreType.DMA((_NBUF,))]
        ),
    )(pred_t, lbl)

    fn_tc, fy_tc = pl.pallas_call(
        _tc_body,
        grid=(_TC_NV,),
        in_specs=[
            pl.BlockSpec((_TC_VB, _B), lambda i: (_V_SC // _TC_VB + i, 0)),
            pl.BlockSpec((1, _B), lambda i: (0, 0)),
        ],
        out_specs=[
            pl.BlockSpec((1, _B), lambda i: (0, 0)),
            pl.BlockSpec((1, _B), lambda i: (0, 0)),
        ],
        out_shape=[
            jax.ShapeDtypeStruct((1, _B), jnp.float32),
            jax.ShapeDtypeStruct((1, _B), jnp.float32),
        ],
    )(pred_t, lbl.reshape(1, _B))

    out = pl.pallas_call(
        _combine_body,
        out_shape=jax.ShapeDtypeStruct((1, 1), jnp.float32),
    )(fn_sc.reshape(1, _B), fy_sc.reshape(1, _B), fn_tc, fy_tc)
    return out[0, 0]


def kernel(prediction, label):
    return _sc_loss(prediction.T, label.astype(jnp.int32))


# speedup vs baseline: 1.0820x; 1.0136x over previous
import jax
import jax.numpy as jnp
from jax import lax
from jax.experimental import pallas as pl
from jax.experimental.pallas import tpu as pltpu
from jax.experimental.pallas import tpu_sc as plsc

_NEG_INF = -10.0 ** 10
_B = 4096
_V = 100000
_SUB = 8
_LANES = 16
_GROUPS = 8
_WCOLS = 128
_NUM_CORES = 2
_NUM_SUBCORES = 16
_NW = _NUM_CORES * _NUM_SUBCORES
_NB_SLOT = 25
_SLOT_ROWS = _NB_SLOT * _SUB
_V_SC = 54400
_V_TC = _V - _V_SC
_NSLOT = _V_SC // _SLOT_ROWS
_NBUF = 4
_NROUND = _NSLOT // _NBUF
_TC_VB = 400
_TC_NV = _V_TC // _TC_VB

_GATHER_1D = lax.GatherDimensionNumbers(
    offset_dims=(), collapsed_slice_dims=(0,), start_index_map=(0,))


def _permute(x, idx):
    return lax.gather(
        x, idx.reshape(_LANES, 1), dimension_numbers=_GATHER_1D,
        slice_sizes=(1,), mode=lax.GatherScatterMode.PROMISE_IN_BOUNDS)


def _xlane_max(x, lane):
    for s in (8, 4, 2, 1):
        x = jnp.maximum(x, _permute(x, lane ^ s))
    return x


def _sc_body(pred_ref, label_ref, fn_ref, fy_ref, label_v, accb, fyb,
             b0, b1, b2, b3, sems):
    bufs = (b0, b1, b2, b3)
    wid = lax.axis_index("s") * _NUM_CORES + lax.axis_index("c")
    col0 = pl.multiple_of(wid * _WCOLS, _WCOLS)

    pltpu.sync_copy(label_ref.at[pl.ds(col0, _WCOLS)], label_v)
    for g in range(_GROUPS):
        accb[pl.ds(g * _LANES, _LANES)] = jnp.full(
            (_LANES,), _NEG_INF, jnp.float32)
        fyb[pl.ds(g * _LANES, _LANES)] = jnp.full(
            (_LANES,), _NEG_INF, jnp.float32)

    def start_slot(s, u):
        pltpu.async_copy(
            pred_ref.at[pl.ds(pl.multiple_of(s * _SLOT_ROWS, _SUB),
                              _SLOT_ROWS),
                        pl.ds(col0, _WCOLS)],
            bufs[u], sems.at[u])

    def wait_slot(u):
        pltpu.make_async_copy(
            pred_ref.at[pl.ds(0, _SLOT_ROWS), pl.ds(0, _WCOLS)],
            bufs[u], sems.at[u]).wait()

    for u in range(_NBUF):
        start_slot(u, u)

    def process_slot(s, u):
        lane = lax.broadcasted_iota(jnp.int32, (_LANES,), 0)
        negv = jnp.full((_LANES,), _NEG_INF, jnp.float32)
        wait_slot(u)
        buf = bufs[u]
        base = s * _SLOT_ROWS
        lblv = [label_v[pl.ds(g * _LANES, _LANES)] for g in range(_GROUPS)]

        hitv = jnp.zeros((_LANES,), jnp.float32)
        for g in range(_GROUPS):
            loc = lblv[g] - base
            inb = (loc >= 0) & (loc < _SLOT_ROWS)
            hitv = hitv + jnp.where(inb, 1.0, 0.0)
        hit = _xlane_max(hitv, lane)[0] > 0.5

        @pl.when(jnp.logical_not(hit))
        def _():
            accs = [accb[pl.ds(g * _LANES, _LANES)] for g in range(_GROUPS)]

            @plsc.parallel_loop(0, _SLOT_ROWS, 1, unroll=2,
                                carry=tuple(accs))
            def red(row, acc):
                return tuple(
                    jnp.maximum(a, buf[row, pl.ds(g * _LANES, _LANES)])
                    for g, a in enumerate(acc))

            for g in range(_GROUPS):
                accb[pl.ds(g * _LANES, _LANES)] = red[g]

        @pl.when(hit)
        def _():
            accs = [accb[pl.ds(g * _LANES, _LANES)] for g in range(_GROUPS)]
            fys = [fyb[pl.ds(g * _LANES, _LANES)] for g in range(_GROUPS)]

            @plsc.parallel_loop(0, _SLOT_ROWS, 1,
                                carry=tuple(accs) + tuple(fys))
            def red(row, c):
                out = list(c[:_GROUPS])
                fy = list(c[_GROUPS:])
                rowv = jnp.full((_LANES,), base + row, jnp.int32)
                for g in range(_GROUPS):
                    x = buf[row, pl.ds(g * _LANES, _LANES)]
                    isl = lblv[g] == rowv
                    out[g] = jnp.maximum(out[g], jnp.where(isl, negv, x))
                    fy[g] = jnp.maximum(fy[g], jnp.where(isl, x, negv))
                return tuple(out) + tuple(fy)

            for g in range(_GROUPS):
                accb[pl.ds(g * _LANES, _LANES)] = red[g]
                fyb[pl.ds(g * _LANES, _LANES)] = red[_GROUPS + g]

    def round_body(ri, carry):
        for u in range(_NBUF):
            s = ri * _NBUF + u
            process_slot(s, u)

            @pl.when(s < _NSLOT - _NBUF)
            def _():
                start_slot(s + _NBUF, u)
        return carry

    lax.fori_loop(0, _NROUND, round_body, 0)

    pltpu.sync_copy(accb, fn_ref.at[wid])
    pltpu.sync_copy(fyb, fy_ref.at[wid])


def _tc_body(pred_ref, lbl_ref, fn_ref, fy_ref):
    i = pl.program_id(0)
    x = pred_ref[...]
    lbl = lbl_ref[...]
    rows = (lax.broadcasted_iota(jnp.int32, (_TC_VB, _B), 0)
            + (_V_SC + i * _TC_VB))
    isl = rows == lbl
    fn = jnp.max(jnp.where(isl, _NEG_INF, x), axis=0, keepdims=True)
    fy = jnp.max(jnp.where(isl, x, _NEG_INF), axis=0, keepdims=True)

    @pl.when(i == 0)
    def _():
        fn_ref[...] = fn
        fy_ref[...] = fy

    @pl.when(i > 0)
    def _():
        fn_ref[...] = jnp.maximum(fn_ref[...], fn)
        fy_ref[...] = jnp.maximum(fy_ref[...], fy)


def _combine_body(fn_sc_ref, fy_sc_ref, fn_tc_ref, fy_tc_ref, out_ref):
    fn = jnp.maximum(fn_sc_ref[...], fn_tc_ref[...])
    fy = jnp.maximum(fy_sc_ref[...], fy_tc_ref[...])
    l = jnp.maximum(2.0 - fy, 0.0) + jnp.maximum(1.0 + fn, 0.0)
    out_ref[...] = jnp.sum(l).reshape(1, 1) * (1.0 / _B)


@jax.jit
def _sc_loss(pred_t, lbl):
    mesh = plsc.VectorSubcoreMesh(
        core_axis_name="c", subcore_axis_name="s",
        num_cores=_NUM_CORES, num_subcores=_NUM_SUBCORES)
    fn_sc, fy_sc = pl.kernel(
        _sc_body,
        out_type=(jax.ShapeDtypeStruct((_NW, _WCOLS), jnp.float32),
                  jax.ShapeDtypeStruct((_NW, _WCOLS), jnp.float32)),
        mesh=mesh,
        compiler_params=pltpu.CompilerParams(use_tc_tiling_on_sc=True),
        scratch_types=(
            [pltpu.VMEM((_WCOLS,), jnp.int32),
             pltpu.VMEM((_WCOLS,), jnp.float32),
             pltpu.VMEM((_WCOLS,), jnp.float32)]
            + [pltpu.VMEM((_SLOT_ROWS, _WCOLS), jnp.float32)] * _NBUF
            + [pltpu.SemaphoreType.DMA((_NBUF,))]
        ),
    )(pred_t, lbl)

    fn_tc, fy_tc = pl.pallas_call(
        _tc_body,
        grid=(_TC_NV,),
        in_specs=[
            pl.BlockSpec((_TC_VB, _B), lambda i: (_V_SC // _TC_VB + i, 0)),
            pl.BlockSpec((1, _B), lambda i: (0, 0)),
        ],
        out_specs=[
            pl.BlockSpec((1, _B), lambda i: (0, 0)),
            pl.BlockSpec((1, _B), lambda i: (0, 0)),
        ],
        out_shape=[
            jax.ShapeDtypeStruct((1, _B), jnp.float32),
            jax.ShapeDtypeStruct((1, _B), jnp.float32),
        ],
    )(pred_t, lbl.reshape(1, _B))

    out = pl.pallas_call(
        _combine_body,
        out_shape=jax.ShapeDtypeStruct((1, 1), jnp.float32),
    )(fn_sc.reshape(1, _B), fy_sc.reshape(1, _B), fn_tc, fy_tc)
    return out[0, 0]


def kernel(prediction, label):
    return _sc_loss(prediction.T, label.astype(jnp.int32))
